# baseline (speedup 1.0000x reference)
import functools

import jax
import jax.numpy as jnp
from jax import lax
from jax.experimental import pallas as pl
from jax.experimental.pallas import tpu as pltpu
from jax.experimental.pallas import tpu_sc as plsc

_N = 10000
_E = 320000
_H = 16
_M = 64
_O = 27

_NC = 2
_NS = 16
_NW = _NC * _NS
_EPT = _E // _NW
_CH = 80
_NCHUNK = _EPT // _CH
_K = 25
_S = _NCHUNK // _K
_NPAD = 10240
_RPT = _NPAD // _NS
_NPK = _NPAD // 8
_NROW = _N // 8

_mesh = plsc.VectorSubcoreMesh(core_axis_name="c", subcore_axis_name="s")
_sc_params = pltpu.CompilerParams(use_tc_tiling_on_sc=False,
                                  needs_layout_passes=False)


def _newton_rsqrt(d):
    xi = plsc.bitcast(d, jnp.int32)
    y = plsc.bitcast(jnp.int32(0x5F3759DF) - lax.shift_right_logical(xi, 1),
                     jnp.float32)
    for _ in range(3):
        y = y * (1.5 - 0.5 * d * y * y)
    return y


def _scale_rows(rows_ref, rinv_ref):
    @pl.loop(0, _RPT // 16)
    def _(blk):
        rv = rinv_ref.at[pl.ds(blk * 16, 16)][...]
        for b in range(16):
            i = blk * 16 + b
            idxv = jnp.full((16,), b, jnp.int32)
            sc = rv.at[idxv].get(mode='promise_in_bounds')
            rows_ref.at[i][...] = rows_ref.at[i][...] * sc


@functools.partial(
    pl.kernel,
    out_type=jax.ShapeDtypeStruct((_NC, _NPAD), jnp.float32),
    mesh=_mesh,
    scratch_types=[
        pltpu.VMEM((_NCHUNK, _CH), jnp.int32),
        pltpu.VMEM((_CH,), jnp.float32),
        pltpu.VMEM((_RPT,), jnp.float32),
        pltpu.VMEM_SHARED((_NPAD,), jnp.float32),
        pltpu.SemaphoreType.DMA,
    ],
    compiler_params=_sc_params,
)
def _deg_kernel(edges_hbm, out_hbm, idx_v, ones_v, stage_v, deg_sh, sem):
    c = lax.axis_index("c")
    s = lax.axis_index("s")
    wid = c * _NS + s
    pltpu.sync_copy(edges_hbm.at[1, wid], idx_v)

    @pl.loop(0, _CH // 16)
    def _(i):
        ones_v.at[pl.ds(i * 16, 16)][...] = jnp.ones((16,), jnp.float32)

    @pl.loop(0, _RPT // 16)
    def _(i):
        stage_v.at[pl.ds(i * 16, 16)][...] = jnp.zeros((16,), jnp.float32)

    pltpu.sync_copy(stage_v, deg_sh.at[pl.ds(s * _RPT, _RPT)])
    plsc.subcore_barrier()

    for ss in range(_S):
        @pl.loop(0, _K)
        def _(j):
            q = ss * _K + j
            pltpu.async_copy(ones_v, deg_sh.at[idx_v.at[q]], sem, add=True)

        @pl.loop(0, _K)
        def _(j):
            q = ss * _K + j
            pltpu.make_async_copy(ones_v, deg_sh.at[idx_v.at[q]], sem).wait()

    plsc.subcore_barrier()
    pltpu.sync_copy(deg_sh.at[pl.ds(s * _RPT, _RPT)], stage_v)
    pltpu.sync_copy(stage_v, out_hbm.at[c, pl.ds(s * _RPT, _RPT)])


@functools.partial(
    pl.kernel,
    out_type=[
        jax.ShapeDtypeStruct((_NC, _NS, _RPT, _H), jnp.float32),
        jax.ShapeDtypeStruct((_NC, _NPAD, _H), jnp.float32),
    ],
    mesh=_mesh,
    scratch_types=[
        pltpu.VMEM((_NCHUNK, _CH), jnp.int32),
        pltpu.VMEM((_NCHUNK, _CH), jnp.int32),
        pltpu.VMEM((2, _K * _CH, _H), jnp.float32),
        pltpu.VMEM((_RPT, _H), jnp.float32),
        pltpu.VMEM((_RPT,), jnp.float32),
        pltpu.VMEM((_RPT,), jnp.float32),
        pltpu.VMEM((_RPT,), jnp.float32),
        pltpu.VMEM_SHARED((_NPAD, _H), jnp.float32),
        pltpu.SemaphoreType.DMA,
        pltpu.SemaphoreType.DMA,
        pltpu.SemaphoreType.DMA,
        pltpu.SemaphoreType.DMA,
    ],
    compiler_params=_sc_params,
)
def _main_kernel(y_hbm, edges_hbm, degp_hbm, out_hbm, ys_hbm,
                 src_v, dst_v, rows_v, stage_v, dga_v, dgb_v, rinv_v,
                 agg_sh, gsem0, gsem1, ssem0, ssem1):
    c = lax.axis_index("c")
    s = lax.axis_index("s")
    wid = c * _NS + s
    pltpu.sync_copy(edges_hbm.at[0, wid], src_v)
    pltpu.sync_copy(edges_hbm.at[1, wid], dst_v)
    pltpu.sync_copy(degp_hbm.at[0, pl.ds(s * _RPT, _RPT)], dga_v)
    pltpu.sync_copy(degp_hbm.at[1, pl.ds(s * _RPT, _RPT)], dgb_v)

    @pl.loop(0, _RPT // 16)
    def _(i):
        sl = pl.ds(i * 16, 16)
        d = jnp.maximum(dga_v.at[sl][...] + dgb_v.at[sl][...], 1.0)
        rinv_v.at[sl][...] = _newton_rsqrt(d)

    @pl.loop(0, _RPT)
    def _(i):
        stage_v.at[i][...] = jnp.zeros((16,), jnp.float32)

    pltpu.sync_copy(stage_v, agg_sh.at[pl.ds(s * _RPT, _RPT), :])

    pltpu.sync_copy(y_hbm.at[pl.ds(s * _RPT, _RPT)], stage_v)
    _scale_rows(stage_v, rinv_v)
    pltpu.sync_copy(stage_v, ys_hbm.at[c, pl.ds(s * _RPT, _RPT), :])
    plsc.subcore_barrier()

    gsems = (gsem0, gsem1)
    ssems = (ssem0, ssem1)

    def fire_gathers(ss):
        st = ss % 2

        @pl.loop(0, _K)
        def _(j):
            q = ss * _K + j
            pltpu.async_copy(ys_hbm.at[c].at[src_v.at[q]],
                             rows_v.at[st].at[pl.ds(j * _CH, _CH), :],
                             gsems[st])

    def drain_gathers(ss):
        st = ss % 2

        @pl.loop(0, _K)
        def _(j):
            q = ss * _K + j
            pltpu.make_async_copy(ys_hbm.at[c].at[src_v.at[q]],
                                  rows_v.at[st].at[pl.ds(j * _CH, _CH), :],
                                  gsems[st]).wait()

    def fire_scatters(ss):
        st = ss % 2

        @pl.loop(0, _K)
        def _(j):
            q = ss * _K + j
            pltpu.async_copy(rows_v.at[st].at[pl.ds(j * _CH, _CH), :],
                             agg_sh.at[dst_v.at[q]], ssems[st], add=True)

    def drain_scatters(ss):
        st = ss % 2

        @pl.loop(0, _K)
        def _(j):
            q = ss * _K + j
            pltpu.make_async_copy(rows_v.at[st].at[pl.ds(j * _CH, _CH), :],
                                  agg_sh.at[dst_v.at[q]], ssems[st]).wait()

    fire_gathers(0)
    for ss in range(_S):
        drain_gathers(ss)
        if ss + 1 < _S:
            fire_gathers(ss + 1)
        fire_scatters(ss)
        drain_scatters(ss)

    plsc.subcore_barrier()
    pltpu.sync_copy(agg_sh.at[pl.ds(s * _RPT, _RPT), :], stage_v)
    _scale_rows(stage_v, rinv_v)
    pltpu.sync_copy(stage_v, out_hbm.at[c, s])


def _prep_body(x_ref, w_ref, b_ref, y_ref):
    y = jnp.dot(x_ref[...], w_ref[...], preferred_element_type=jnp.float32)
    y = jnp.maximum(y + b_ref[...], 0.0)
    y_ref[pl.ds(_NPK - 32, 32), :] = jnp.zeros((32, 128), jnp.float32)
    y_ref[pl.ds(0, _NROW), :] = y


def _final_body(p_ref, w_ref, b_ref, wo_ref, bo_ref, out_ref):
    p = p_ref[...]
    z = (p[0] + p[1]).reshape(_NPK, 128)
    t = jnp.dot(z, w_ref[...], preferred_element_type=jnp.float32)
    t = jnp.maximum(t + b_ref[...], 0.0)
    rowid = lax.broadcasted_iota(jnp.int32, (_NPK, 1), 0)
    t = jnp.where(rowid < _NROW, t, 0.0)
    ro512 = jnp.sum(t, axis=0, keepdims=True)
    ro = ro512[:, 0:_M]
    for a in range(1, 8):
        ro = ro + ro512[:, a * _M:(a + 1) * _M]
    out_ref[...] = (
        jnp.dot(ro, wo_ref[...], preferred_element_type=jnp.float32)
        + bo_ref[...])


@jax.jit
def _run(node_features, edge_index, W_node, b_node, W_gcn, b_gcn, W_out,
         b_out):
    edges = edge_index.reshape(2, _NW, _NCHUNK, _CH)
    x2 = node_features.reshape(_NROW, 8 * 128)
    eye8 = jnp.eye(8, dtype=jnp.float32)
    W_bd = jnp.kron(eye8, W_node)
    bn_t = jnp.tile(b_node, 8)[None]
    W_bd2 = jnp.kron(eye8, W_gcn)
    bg_t = jnp.tile(b_gcn, 8)[None]

    degp = _deg_kernel(edges)

    y = pl.pallas_call(
        _prep_body,
        out_shape=jax.ShapeDtypeStruct((_NPK, 128), jnp.float32),
    )(x2, W_bd, bn_t)

    part, _ = _main_kernel(y.reshape(_NPAD, _H), edges, degp)
    part = part.reshape(_NC, _NS, _RPT // 8, 128)

    out = pl.pallas_call(
        _final_body,
        out_shape=jax.ShapeDtypeStruct((1, _O), jnp.float32),
    )(part, W_bd2, bg_t, W_out, b_out.reshape(1, _O))
    return out


def kernel(node_features, edge_features, edge_index, W_node, b_node, W_edge,
           b_edge, W_gcn, b_gcn, W_out, b_out):
    return _run(node_features, edge_index, W_node, b_node, W_gcn, b_gcn,
                W_out, b_out)


# speedup vs baseline: 1.0700x; 1.0700x over previous
import functools

import jax
import jax.numpy as jnp
from jax import lax
from jax.experimental import pallas as pl
from jax.experimental.pallas import tpu as pltpu
from jax.experimental.pallas import tpu_sc as plsc

_N = 10000
_E = 320000
_H = 16
_M = 64
_O = 27

_NC = 2
_NS = 16
_NW = _NC * _NS
_EPT = _E // _NW
_CH = 400
_NCHUNK = _EPT // _CH
_K = 5
_S = _NCHUNK // _K
_NPAD = 10240
_RPT = _NPAD // _NS
_NPK = _NPAD // 8
_NROW = _N // 8

_mesh = plsc.VectorSubcoreMesh(core_axis_name="c", subcore_axis_name="s")
_sc_params = pltpu.CompilerParams(use_tc_tiling_on_sc=False,
                                  needs_layout_passes=False)


def _newton_rsqrt(d):
    xi = plsc.bitcast(d, jnp.int32)
    y = plsc.bitcast(jnp.int32(0x5F3759DF) - lax.shift_right_logical(xi, 1),
                     jnp.float32)
    for _ in range(3):
        y = y * (1.5 - 0.5 * d * y * y)
    return y


def _scale_rows(rows_ref, rinv_ref):
    @pl.loop(0, _RPT // 16)
    def _(blk):
        rv = rinv_ref.at[pl.ds(blk * 16, 16)][...]
        for b in range(16):
            i = blk * 16 + b
            idxv = jnp.full((16,), b, jnp.int32)
            sc = rv.at[idxv].get(mode='promise_in_bounds')
            rows_ref.at[i][...] = rows_ref.at[i][...] * sc


@functools.partial(
    pl.kernel,
    out_type=jax.ShapeDtypeStruct((_NC, _NPAD), jnp.float32),
    mesh=_mesh,
    scratch_types=[
        pltpu.VMEM((_NCHUNK, _CH), jnp.int32),
        pltpu.VMEM((_CH,), jnp.float32),
        pltpu.VMEM((_RPT,), jnp.float32),
        pltpu.VMEM_SHARED((_NPAD,), jnp.float32),
        pltpu.SemaphoreType.DMA,
    ],
    compiler_params=_sc_params,
)
def _deg_kernel(edges_hbm, out_hbm, idx_v, ones_v, stage_v, deg_sh, sem):
    c = lax.axis_index("c")
    s = lax.axis_index("s")
    wid = c * _NS + s
    pltpu.sync_copy(edges_hbm.at[1, wid], idx_v)

    @pl.loop(0, _CH // 16)
    def _(i):
        ones_v.at[pl.ds(i * 16, 16)][...] = jnp.ones((16,), jnp.float32)

    @pl.loop(0, _RPT // 16)
    def _(i):
        stage_v.at[pl.ds(i * 16, 16)][...] = jnp.zeros((16,), jnp.float32)

    pltpu.sync_copy(stage_v, deg_sh.at[pl.ds(s * _RPT, _RPT)])
    plsc.subcore_barrier()

    for ss in range(_S):
        @pl.loop(0, _K)
        def _(j):
            q = ss * _K + j
            pltpu.async_copy(ones_v, deg_sh.at[idx_v.at[q]], sem, add=True)

        @pl.loop(0, _K)
        def _(j):
            q = ss * _K + j
            pltpu.make_async_copy(ones_v, deg_sh.at[idx_v.at[q]], sem).wait()

    plsc.subcore_barrier()
    pltpu.sync_copy(deg_sh.at[pl.ds(s * _RPT, _RPT)], stage_v)
    pltpu.sync_copy(stage_v, out_hbm.at[c, pl.ds(s * _RPT, _RPT)])


@functools.partial(
    pl.kernel,
    out_type=jax.ShapeDtypeStruct((_NC, _NS, _RPT, _H), jnp.float32),
    mesh=_mesh,
    scratch_types=[
        pltpu.VMEM((_NCHUNK, _CH), jnp.int32),
        pltpu.VMEM((_NCHUNK, _CH), jnp.int32),
        pltpu.VMEM((2, _K * _CH, _H), jnp.float32),
        pltpu.VMEM((_RPT, _H), jnp.float32),
        pltpu.VMEM((_RPT,), jnp.float32),
        pltpu.VMEM((_RPT,), jnp.float32),
        pltpu.VMEM((_RPT,), jnp.float32),
        pltpu.VMEM_SHARED((_NPAD, _H), jnp.float32),
        pltpu.VMEM_SHARED((_NPAD, _H), jnp.float32),
        pltpu.SemaphoreType.DMA,
        pltpu.SemaphoreType.DMA,
        pltpu.SemaphoreType.DMA,
        pltpu.SemaphoreType.DMA,
    ],
    compiler_params=_sc_params,
)
def _main_kernel(y_hbm, edges_hbm, degp_hbm, out_hbm,
                 src_v, dst_v, rows_v, stage_v, dga_v, dgb_v, rinv_v,
                 y_sh, agg_sh, gsem0, gsem1, ssem0, ssem1):
    c = lax.axis_index("c")
    s = lax.axis_index("s")
    wid = c * _NS + s
    pltpu.sync_copy(edges_hbm.at[0, wid], src_v)
    pltpu.sync_copy(edges_hbm.at[1, wid], dst_v)
    pltpu.sync_copy(degp_hbm.at[0, pl.ds(s * _RPT, _RPT)], dga_v)
    pltpu.sync_copy(degp_hbm.at[1, pl.ds(s * _RPT, _RPT)], dgb_v)

    @pl.loop(0, _RPT // 16)
    def _(i):
        sl = pl.ds(i * 16, 16)
        d = jnp.maximum(dga_v.at[sl][...] + dgb_v.at[sl][...], 1.0)
        rinv_v.at[sl][...] = _newton_rsqrt(d)

    @pl.loop(0, _RPT)
    def _(i):
        stage_v.at[i][...] = jnp.zeros((16,), jnp.float32)

    pltpu.sync_copy(stage_v, agg_sh.at[pl.ds(s * _RPT, _RPT), :])

    pltpu.sync_copy(y_hbm.at[pl.ds(s * _RPT, _RPT)], stage_v)
    _scale_rows(stage_v, rinv_v)
    pltpu.sync_copy(stage_v, y_sh.at[pl.ds(s * _RPT, _RPT), :])
    plsc.subcore_barrier()

    gsems = (gsem0, gsem1)
    ssems = (ssem0, ssem1)

    def fire_gathers(ss):
        st = ss % 2

        @pl.loop(0, _K)
        def _(j):
            q = ss * _K + j
            pltpu.async_copy(y_sh.at[src_v.at[q]],
                             rows_v.at[st].at[pl.ds(j * _CH, _CH), :],
                             gsems[st])

    def drain_gathers(ss):
        st = ss % 2

        @pl.loop(0, _K)
        def _(j):
            q = ss * _K + j
            pltpu.make_async_copy(y_sh.at[src_v.at[q]],
                                  rows_v.at[st].at[pl.ds(j * _CH, _CH), :],
                                  gsems[st]).wait()

    def fire_scatters(ss):
        st = ss % 2

        @pl.loop(0, _K)
        def _(j):
            q = ss * _K + j
            pltpu.async_copy(rows_v.at[st].at[pl.ds(j * _CH, _CH), :],
                             agg_sh.at[dst_v.at[q]], ssems[st], add=True)

    def drain_scatters(ss):
        st = ss % 2

        @pl.loop(0, _K)
        def _(j):
            q = ss * _K + j
            pltpu.make_async_copy(rows_v.at[st].at[pl.ds(j * _CH, _CH), :],
                                  agg_sh.at[dst_v.at[q]], ssems[st]).wait()

    fire_gathers(0)
    for ss in range(_S):
        drain_gathers(ss)
        if ss + 1 < _S:
            fire_gathers(ss + 1)
        fire_scatters(ss)
        drain_scatters(ss)

    plsc.subcore_barrier()
    pltpu.sync_copy(agg_sh.at[pl.ds(s * _RPT, _RPT), :], stage_v)
    _scale_rows(stage_v, rinv_v)
    pltpu.sync_copy(stage_v, out_hbm.at[c, s])


def _prep_body(x_ref, w_ref, b_ref, y_ref):
    y = jnp.dot(x_ref[...], w_ref[...], preferred_element_type=jnp.float32)
    y = jnp.maximum(y + b_ref[...], 0.0)
    y_ref[pl.ds(_NPK - 32, 32), :] = jnp.zeros((32, 128), jnp.float32)
    y_ref[pl.ds(0, _NROW), :] = y


def _final_body(p_ref, w_ref, b_ref, wo_ref, bo_ref, out_ref):
    p = p_ref[...]
    z = (p[0] + p[1]).reshape(_NPK, 128)
    t = jnp.dot(z, w_ref[...], preferred_element_type=jnp.float32)
    t = jnp.maximum(t + b_ref[...], 0.0)
    rowid = lax.broadcasted_iota(jnp.int32, (_NPK, 1), 0)
    t = jnp.where(rowid < _NROW, t, 0.0)
    ro512 = jnp.sum(t, axis=0, keepdims=True)
    ro = ro512[:, 0:_M]
    for a in range(1, 8):
        ro = ro + ro512[:, a * _M:(a + 1) * _M]
    out_ref[...] = (
        jnp.dot(ro, wo_ref[...], preferred_element_type=jnp.float32)
        + bo_ref[...])


@jax.jit
def _run(node_features, edge_index, W_node, b_node, W_gcn, b_gcn, W_out,
         b_out):
    edges = edge_index.reshape(2, _NW, _NCHUNK, _CH)
    x2 = node_features.reshape(_NROW, 8 * 128)
    eye8 = jnp.eye(8, dtype=jnp.float32)
    W_bd = jnp.kron(eye8, W_node)
    bn_t = jnp.tile(b_node, 8)[None]
    W_bd2 = jnp.kron(eye8, W_gcn)
    bg_t = jnp.tile(b_gcn, 8)[None]

    degp = _deg_kernel(edges)

    y = pl.pallas_call(
        _prep_body,
        out_shape=jax.ShapeDtypeStruct((_NPK, 128), jnp.float32),
    )(x2, W_bd, bn_t)

    part = _main_kernel(y.reshape(_NPAD, _H), edges, degp)
    part = part.reshape(_NC, _NS, _RPT // 8, 128)

    out = pl.pallas_call(
        _final_body,
        out_shape=jax.ShapeDtypeStruct((1, _O), jnp.float32),
    )(part, W_bd2, bg_t, W_out, b_out.reshape(1, _O))
    return out


def kernel(node_features, edge_features, edge_index, W_node, b_node, W_edge,
           b_edge, W_gcn, b_gcn, W_out, b_out):
    return _run(node_features, edge_index, W_node, b_node, W_gcn, b_gcn,
                W_out, b_out)


# speedup vs baseline: 1.0718x; 1.0017x over previous
import functools

import jax
import jax.numpy as jnp
from jax import lax
from jax.experimental import pallas as pl
from jax.experimental.pallas import tpu as pltpu
from jax.experimental.pallas import tpu_sc as plsc

_N = 10000
_E = 320000
_H = 16
_M = 64
_O = 27

_NC = 2
_NS = 16
_NW = _NC * _NS
_EPT = _E // _NW
_CH = 2000
_NCHUNK = _EPT // _CH
_K = 1
_S = _NCHUNK // _K
_NPAD = 10240
_RPT = _NPAD // _NS
_NPK = _NPAD // 8
_NROW = _N // 8

_mesh = plsc.VectorSubcoreMesh(core_axis_name="c", subcore_axis_name="s")
_sc_params = pltpu.CompilerParams(use_tc_tiling_on_sc=False,
                                  needs_layout_passes=False)


def _newton_rsqrt(d):
    xi = plsc.bitcast(d, jnp.int32)
    y = plsc.bitcast(jnp.int32(0x5F3759DF) - lax.shift_right_logical(xi, 1),
                     jnp.float32)
    for _ in range(3):
        y = y * (1.5 - 0.5 * d * y * y)
    return y


def _scale_rows(rows_ref, rinv_ref):
    @pl.loop(0, _RPT // 16)
    def _(blk):
        rv = rinv_ref.at[pl.ds(blk * 16, 16)][...]
        for b in range(16):
            i = blk * 16 + b
            idxv = jnp.full((16,), b, jnp.int32)
            sc = rv.at[idxv].get(mode='promise_in_bounds')
            rows_ref.at[i][...] = rows_ref.at[i][...] * sc


@functools.partial(
    pl.kernel,
    out_type=jax.ShapeDtypeStruct((_NC, _NPAD), jnp.float32),
    mesh=_mesh,
    scratch_types=[
        pltpu.VMEM((_NCHUNK, _CH), jnp.int32),
        pltpu.VMEM((_CH,), jnp.float32),
        pltpu.VMEM((_RPT,), jnp.float32),
        pltpu.VMEM_SHARED((_NPAD,), jnp.float32),
        pltpu.SemaphoreType.DMA,
    ],
    compiler_params=_sc_params,
)
def _deg_kernel(edges_hbm, out_hbm, idx_v, ones_v, stage_v, deg_sh, sem):
    c = lax.axis_index("c")
    s = lax.axis_index("s")
    wid = c * _NS + s
    pltpu.sync_copy(edges_hbm.at[1, wid], idx_v)

    @pl.loop(0, _CH // 16)
    def _(i):
        ones_v.at[pl.ds(i * 16, 16)][...] = jnp.ones((16,), jnp.float32)

    @pl.loop(0, _RPT // 16)
    def _(i):
        stage_v.at[pl.ds(i * 16, 16)][...] = jnp.zeros((16,), jnp.float32)

    pltpu.sync_copy(stage_v, deg_sh.at[pl.ds(s * _RPT, _RPT)])
    plsc.subcore_barrier()

    for ss in range(_S):
        @pl.loop(0, _K)
        def _(j):
            q = ss * _K + j
            pltpu.async_copy(ones_v, deg_sh.at[idx_v.at[q]], sem, add=True)

        @pl.loop(0, _K)
        def _(j):
            q = ss * _K + j
            pltpu.make_async_copy(ones_v, deg_sh.at[idx_v.at[q]], sem).wait()

    plsc.subcore_barrier()
    pltpu.sync_copy(deg_sh.at[pl.ds(s * _RPT, _RPT)], stage_v)
    pltpu.sync_copy(stage_v, out_hbm.at[c, pl.ds(s * _RPT, _RPT)])


@functools.partial(
    pl.kernel,
    out_type=jax.ShapeDtypeStruct((_NC, _NS, _RPT, _H), jnp.float32),
    mesh=_mesh,
    scratch_types=[
        pltpu.VMEM((_NCHUNK, _CH), jnp.int32),
        pltpu.VMEM((_NCHUNK, _CH), jnp.int32),
        pltpu.VMEM((2, _K * _CH, _H), jnp.float32),
        pltpu.VMEM((_RPT, _H), jnp.float32),
        pltpu.VMEM((_RPT,), jnp.float32),
        pltpu.VMEM((_RPT,), jnp.float32),
        pltpu.VMEM((_RPT,), jnp.float32),
        pltpu.VMEM_SHARED((_NPAD, _H), jnp.float32),
        pltpu.VMEM_SHARED((_NPAD, _H), jnp.float32),
        pltpu.SemaphoreType.DMA,
        pltpu.SemaphoreType.DMA,
        pltpu.SemaphoreType.DMA,
        pltpu.SemaphoreType.DMA,
    ],
    compiler_params=_sc_params,
)
def _main_kernel(y_hbm, edges_hbm, degp_hbm, out_hbm,
                 src_v, dst_v, rows_v, stage_v, dga_v, dgb_v, rinv_v,
                 y_sh, agg_sh, gsem0, gsem1, ssem0, ssem1):
    c = lax.axis_index("c")
    s = lax.axis_index("s")
    wid = c * _NS + s
    pltpu.sync_copy(edges_hbm.at[0, wid], src_v)
    pltpu.sync_copy(edges_hbm.at[1, wid], dst_v)
    pltpu.sync_copy(degp_hbm.at[0, pl.ds(s * _RPT, _RPT)], dga_v)
    pltpu.sync_copy(degp_hbm.at[1, pl.ds(s * _RPT, _RPT)], dgb_v)

    @pl.loop(0, _RPT // 16)
    def _(i):
        sl = pl.ds(i * 16, 16)
        d = jnp.maximum(dga_v.at[sl][...] + dgb_v.at[sl][...], 1.0)
        rinv_v.at[sl][...] = _newton_rsqrt(d)

    @pl.loop(0, _RPT)
    def _(i):
        stage_v.at[i][...] = jnp.zeros((16,), jnp.float32)

    pltpu.sync_copy(stage_v, agg_sh.at[pl.ds(s * _RPT, _RPT), :])

    pltpu.sync_copy(y_hbm.at[pl.ds(s * _RPT, _RPT)], stage_v)
    _scale_rows(stage_v, rinv_v)
    pltpu.sync_copy(stage_v, y_sh.at[pl.ds(s * _RPT, _RPT), :])
    plsc.subcore_barrier()

    gsems = (gsem0, gsem1)
    ssems = (ssem0, ssem1)

    def fire_gathers(ss):
        st = ss % 2

        @pl.loop(0, _K)
        def _(j):
            q = ss * _K + j
            pltpu.async_copy(y_sh.at[src_v.at[q]],
                             rows_v.at[st].at[pl.ds(j * _CH, _CH), :],
                             gsems[st])

    def drain_gathers(ss):
        st = ss % 2

        @pl.loop(0, _K)
        def _(j):
            q = ss * _K + j
            pltpu.make_async_copy(y_sh.at[src_v.at[q]],
                                  rows_v.at[st].at[pl.ds(j * _CH, _CH), :],
                                  gsems[st]).wait()

    def fire_scatters(ss):
        st = ss % 2

        @pl.loop(0, _K)
        def _(j):
            q = ss * _K + j
            pltpu.async_copy(rows_v.at[st].at[pl.ds(j * _CH, _CH), :],
                             agg_sh.at[dst_v.at[q]], ssems[st], add=True)

    def drain_scatters(ss):
        st = ss % 2

        @pl.loop(0, _K)
        def _(j):
            q = ss * _K + j
            pltpu.make_async_copy(rows_v.at[st].at[pl.ds(j * _CH, _CH), :],
                                  agg_sh.at[dst_v.at[q]], ssems[st]).wait()

    fire_gathers(0)
    for ss in range(_S):
        drain_gathers(ss)
        if ss + 1 < _S:
            fire_gathers(ss + 1)
        fire_scatters(ss)
        drain_scatters(ss)

    plsc.subcore_barrier()
    pltpu.sync_copy(agg_sh.at[pl.ds(s * _RPT, _RPT), :], stage_v)
    _scale_rows(stage_v, rinv_v)
    pltpu.sync_copy(stage_v, out_hbm.at[c, s])


def _prep_body(x_ref, w_ref, b_ref, y_ref):
    y = jnp.dot(x_ref[...], w_ref[...], preferred_element_type=jnp.float32)
    y = jnp.maximum(y + b_ref[...], 0.0)
    y_ref[pl.ds(_NPK - 32, 32), :] = jnp.zeros((32, 128), jnp.float32)
    y_ref[pl.ds(0, _NROW), :] = y


def _final_body(p_ref, w_ref, b_ref, wo_ref, bo_ref, out_ref):
    p = p_ref[...]
    z = (p[0] + p[1]).reshape(_NPK, 128)
    t = jnp.dot(z, w_ref[...], preferred_element_type=jnp.float32)
    t = jnp.maximum(t + b_ref[...], 0.0)
    rowid = lax.broadcasted_iota(jnp.int32, (_NPK, 1), 0)
    t = jnp.where(rowid < _NROW, t, 0.0)
    ro512 = jnp.sum(t, axis=0, keepdims=True)
    ro = ro512[:, 0:_M]
    for a in range(1, 8):
        ro = ro + ro512[:, a * _M:(a + 1) * _M]
    out_ref[...] = (
        jnp.dot(ro, wo_ref[...], preferred_element_type=jnp.float32)
        + bo_ref[...])


@jax.jit
def _run(node_features, edge_index, W_node, b_node, W_gcn, b_gcn, W_out,
         b_out):
    edges = edge_index.reshape(2, _NW, _NCHUNK, _CH)
    x2 = node_features.reshape(_NROW, 8 * 128)
    eye8 = jnp.eye(8, dtype=jnp.float32)
    W_bd = jnp.kron(eye8, W_node)
    bn_t = jnp.tile(b_node, 8)[None]
    W_bd2 = jnp.kron(eye8, W_gcn)
    bg_t = jnp.tile(b_gcn, 8)[None]

    degp = _deg_kernel(edges)

    y = pl.pallas_call(
        _prep_body,
        out_shape=jax.ShapeDtypeStruct((_NPK, 128), jnp.float32),
    )(x2, W_bd, bn_t)

    part = _main_kernel(y.reshape(_NPAD, _H), edges, degp)
    part = part.reshape(_NC, _NS, _RPT // 8, 128)

    out = pl.pallas_call(
        _final_body,
        out_shape=jax.ShapeDtypeStruct((1, _O), jnp.float32),
    )(part, W_bd2, bg_t, W_out, b_out.reshape(1, _O))
    return out


def kernel(node_features, edge_features, edge_index, W_node, b_node, W_edge,
           b_edge, W_gcn, b_gcn, W_out, b_out):
    return _run(node_features, edge_index, W_node, b_node, W_gcn, b_gcn,
                W_out, b_out)
